# baseline (speedup 1.0000x reference)
import functools

import jax
import jax.numpy as jnp
import numpy as np
from jax import lax
from jax.experimental import pallas as pl
from jax.experimental.pallas import tpu as pltpu
from jax.experimental.pallas import tpu_sc as plsc

B, T, D = 4, 4096, 128
T4 = T // 4
W = 12
NW_CAND = 26
KB = 512
NB = 1024
NEG = -1e9

_TID = np.array(sorted(int(v) for v in np.linspace(0, T - 1, T4)), dtype=np.int32)
_E = (_TID - 4 * np.arange(T4, dtype=np.int32)).astype(np.int32)


NCOL = 4 * KB + 32


def _select_kernel(e_ref, et_ref, x_ref, x0_ref, x1_ref, x2_ref, x3_ref, out_ref):
    b = pl.program_id(0)
    kb = pl.program_id(1)
    xc = (x0_ref, x1_ref, x2_ref, x3_ref)

    e = e_ref[...]

    base_q = kb * KB
    sel_x = jnp.zeros((KB, D), jnp.float32)
    for c in range(4):
        seg = xc[c][0, pl.ds(base_q, KB), :]
        sel_x = jnp.where(e == c, seg, sel_x)

    KS = 32
    NCS = 4 * KS + 32
    lane_c = lax.broadcasted_iota(jnp.int32, (KS, NCS), 1)
    kl = lax.broadcasted_iota(jnp.int32, (KS, 1), 0)
    s_sub_rows = []
    for ks in range(0, KB, KS):
        sel_sub = sel_x[ks:ks + KS, :]
        n0 = 4 * (base_q + ks) - 16
        a0 = lax.clamp(0, n0, T - NCS)
        cols = x_ref[0, pl.ds(a0, NCS), :]
        S_sub = lax.dot_general(sel_sub, cols, (((1,), (1,)), ((), ())))
        off = n0 - a0
        sw = []
        for w in range(NW_CAND):
            m = lane_c == (4 * kl + (w + 5) + off)
            sw.append(jnp.sum(jnp.where(m, S_sub, 0.0), axis=1, keepdims=True))
        s_sub_rows.append(jnp.concatenate(sw, axis=1))
    Sraw = jnp.concatenate(s_sub_rows, axis=0)

    St = jnp.transpose(Sraw)
    eT = et_ref[...]
    k_lane = lax.broadcasted_iota(jnp.int32, (NW_CAND, KB), 1) + kb * KB
    w_sub = lax.broadcasted_iota(jnp.int32, (NW_CAND, KB), 0)
    jmat = 4 * k_lane + w_sub - 11
    valid = (w_sub >= eT) & (w_sub <= eT + 22) & (jmat >= 0) & (jmat <= T - 1)
    S = jnp.where(valid, St, NEG)

    Sp = S[:, None, :]
    Sq = S[None, :, :]
    wp = lax.broadcasted_iota(jnp.int32, (NW_CAND, NW_CAND, 1), 0)
    wq = lax.broadcasted_iota(jnp.int32, (NW_CAND, NW_CAND, 1), 1)
    beats = (Sp > Sq) | ((Sp == Sq) & (wp < wq))
    rank = jnp.sum(beats.astype(jnp.int32), axis=0)
    keep = rank < W
    pos = jnp.sum((keep[:, None, :] & (wp < wq)).astype(jnp.int32), axis=0)

    t_sub = lax.broadcasted_iota(jnp.int32, (NW_CAND, W, 1), 1)
    onehot = keep[:, None, :] & (pos[:, None, :] == t_sub)
    jabs = jmat + b * T
    acc = jnp.sum(jnp.where(onehot, jabs[:, None, :], 0), axis=0)
    out_ref[...] = acc


def _run_select(e2, et2, xp, xcs):
    grid = (B, T4 // KB)
    qn = xcs[0].shape[1]
    in_specs = [
        pl.BlockSpec((KB, 1), lambda b, kb: (kb, 0)),
        pl.BlockSpec((1, KB), lambda b, kb: (0, kb)),
        pl.BlockSpec((1, T, D), lambda b, kb: (b, 0, 0)),
    ] + [
        pl.BlockSpec((1, qn, D), lambda b, kb: (b, 0, 0))
        for _ in range(4)
    ]
    return pl.pallas_call(
        _select_kernel,
        grid=grid,
        in_specs=in_specs,
        out_specs=pl.BlockSpec((W, KB), lambda b, kb: (0, b * (T4 // KB) + kb)),
        out_shape=jax.ShapeDtypeStruct((W, B * T4), jnp.int32),
    )(e2, et2, xp, *xcs)


_SC_CHUNK = 128
_SC_NC = 2
_SC_NS = 16
_SC_NW = _SC_NC * _SC_NS


def _sc_gather_body(table_hbm, idx_hbm, out_hbm, idx_v, r0, r1, gs0, gs1, ss0, ss1):
    nrows = B * T4 * W
    per_w = nrows // _SC_NW
    nchunk = per_w // _SC_CHUNK
    wid = lax.axis_index("s") * _SC_NC + lax.axis_index("c")
    base = wid * per_w
    bufs, gsems, ssems = (r0, r1), (gs0, gs1), (ss0, ss1)
    scat = [None, None]
    pltpu.sync_copy(idx_hbm.at[pl.ds(base, per_w)], idx_v)
    for ci in range(nchunk):
        s = ci & 1
        if scat[s] is not None:
            scat[s].wait()
        idx_c = idx_v.at[pl.ds(ci * _SC_CHUNK, _SC_CHUNK)]
        pltpu.async_copy(table_hbm.at[idx_c], bufs[s], gsems[s]).wait()
        scat[s] = pltpu.async_copy(
            bufs[s], out_hbm.at[pl.ds(base + ci * _SC_CHUNK, _SC_CHUNK)], ssems[s])
    scat[0].wait()
    scat[1].wait()


def _sc_gather(xflat, idxflat):
    nrows = B * T4 * W
    per_w = nrows // _SC_NW
    mesh = plsc.VectorSubcoreMesh(core_axis_name="c", subcore_axis_name="s")
    k = functools.partial(
        pl.kernel,
        mesh=mesh,
        out_type=jax.ShapeDtypeStruct((nrows, D), jnp.float32),
        scratch_types=[
            pltpu.VMEM((per_w,), jnp.int32),
            pltpu.VMEM((_SC_CHUNK, D), jnp.float32),
            pltpu.VMEM((_SC_CHUNK, D), jnp.float32),
            pltpu.SemaphoreType.DMA,
            pltpu.SemaphoreType.DMA,
            pltpu.SemaphoreType.DMA,
            pltpu.SemaphoreType.DMA,
        ],
    )(_sc_gather_body)
    return k(xflat, idxflat)


def _gru_kernel(f_ref, wih_ref, whh_ref, bih_ref, bhh_ref, out_ref):
    wih = wih_ref[...]
    whh = whh_ref[...]
    bih = bih_ref[...]
    bhh = bhh_ref[...]
    f_all = f_ref[...].reshape(W * NB, D).astype(jnp.bfloat16)
    gi_all = jnp.dot(f_all, wih, preferred_element_type=jnp.float32)
    h = jnp.zeros((NB, D), jnp.float32)
    for t in range(W):
        gi = gi_all[t * NB:(t + 1) * NB, :] + bih
        gh = jnp.dot(h.astype(jnp.bfloat16), whh,
                     preferred_element_type=jnp.float32) + bhh
        r = jax.nn.sigmoid(gi[:, :D] + gh[:, :D])
        z = jax.nn.sigmoid(gi[:, D:2 * D] + gh[:, D:2 * D])
        n = jnp.tanh(gi[:, 2 * D:] + r * gh[:, 2 * D:])
        h = (1.0 - z) * n + z * h
    out_ref[...] = h


def _run_gru(feat, wihT, whhT, bih2, bhh2):
    ntot = B * T4
    grid = (ntot // NB,)
    return pl.pallas_call(
        _gru_kernel,
        grid=grid,
        in_specs=[
            pl.BlockSpec((W, NB, D), lambda nb: (0, nb, 0)),
            pl.BlockSpec((D, 3 * D), lambda nb: (0, 0)),
            pl.BlockSpec((D, 3 * D), lambda nb: (0, 0)),
            pl.BlockSpec((1, 3 * D), lambda nb: (0, 0)),
            pl.BlockSpec((1, 3 * D), lambda nb: (0, 0)),
        ],
        out_specs=pl.BlockSpec((NB, D), lambda nb: (nb, 0)),
        out_shape=jax.ShapeDtypeStruct((ntot, D), jnp.float32),
    )(feat, wihT, whhT, bih2, bhh2)


def kernel(x, W_ih, W_hh, b_ih, b_hh):
    xr = x.reshape(B, T4, 4, D)
    xcs = [xr[:, :, c, :] for c in range(4)]
    e2 = jnp.asarray(_E).reshape(T4, 1)
    et2 = jnp.asarray(_E).reshape(1, T4)

    idx = _run_select(e2, et2, x, xcs)

    idx_t_major = idx.reshape(W * B * T4)
    xflat = x.reshape(B * T, D)
    feat = _sc_gather(xflat, idx_t_major)
    feat = feat.reshape(W, B * T4, D)

    h = _run_gru(feat, W_ih.T.astype(jnp.bfloat16), W_hh.T.astype(jnp.bfloat16),
                 b_ih.reshape(1, 3 * D), b_hh.reshape(1, 3 * D))
    return h.reshape(B, T4, D)


# speedup vs baseline: 1.0849x; 1.0849x over previous
import functools

import jax
import jax.numpy as jnp
import numpy as np
from jax import lax
from jax.experimental import pallas as pl
from jax.experimental.pallas import tpu as pltpu
from jax.experimental.pallas import tpu_sc as plsc

B, T, D = 4, 4096, 128
T4 = T // 4
W = 12
NW_CAND = 26
KB = 512
NB = 1024
NEG = -1e9

_TID = np.array(sorted(int(v) for v in np.linspace(0, T - 1, T4)), dtype=np.int32)
_E = (_TID - 4 * np.arange(T4, dtype=np.int32)).astype(np.int32)


NCOL = 4 * KB + 32


def _select_kernel(e_ref, et_ref, x_ref, x0_ref, x1_ref, x2_ref, x3_ref, out_ref):
    b = pl.program_id(0)
    kb = pl.program_id(1)
    xc = (x0_ref, x1_ref, x2_ref, x3_ref)

    e = e_ref[...]

    base_q = kb * KB
    sel_x = jnp.zeros((KB, D), jnp.float32)
    for c in range(4):
        seg = xc[c][0, pl.ds(base_q, KB), :]
        sel_x = jnp.where(e == c, seg, sel_x)

    KS = 32
    NCS = 4 * KS + 32
    lane_c = lax.broadcasted_iota(jnp.int32, (KS, NCS), 1)
    kl = lax.broadcasted_iota(jnp.int32, (KS, 1), 0)
    s_sub_rows = []
    for ks in range(0, KB, KS):
        sel_sub = sel_x[ks:ks + KS, :]
        n0 = 4 * (base_q + ks) - 16
        a0 = lax.clamp(0, n0, T - NCS)
        cols = x_ref[0, pl.ds(a0, NCS), :]
        S_sub = lax.dot_general(sel_sub, cols, (((1,), (1,)), ((), ())))
        off = n0 - a0
        sw = []
        for w in range(NW_CAND):
            m = lane_c == (4 * kl + (w + 5) + off)
            sw.append(jnp.sum(jnp.where(m, S_sub, 0.0), axis=1, keepdims=True))
        s_sub_rows.append(jnp.concatenate(sw, axis=1))
    Sraw = jnp.concatenate(s_sub_rows, axis=0)

    St = jnp.transpose(Sraw)
    eT = et_ref[...]
    k_lane = lax.broadcasted_iota(jnp.int32, (NW_CAND, KB), 1) + kb * KB
    w_sub = lax.broadcasted_iota(jnp.int32, (NW_CAND, KB), 0)
    jmat = 4 * k_lane + w_sub - 11
    valid = (w_sub >= eT) & (w_sub <= eT + 22) & (jmat >= 0) & (jmat <= T - 1)
    S = jnp.where(valid, St, NEG)

    Sp = S[:, None, :]
    Sq = S[None, :, :]
    wp = lax.broadcasted_iota(jnp.int32, (NW_CAND, NW_CAND, 1), 0)
    wq = lax.broadcasted_iota(jnp.int32, (NW_CAND, NW_CAND, 1), 1)
    beats = (Sp > Sq) | ((Sp == Sq) & (wp < wq))
    rank = jnp.sum(beats.astype(jnp.int32), axis=0)
    keep = rank < W
    pos = jnp.sum((keep[:, None, :] & (wp < wq)).astype(jnp.int32), axis=0)

    t_sub = lax.broadcasted_iota(jnp.int32, (NW_CAND, W, 1), 1)
    onehot = keep[:, None, :] & (pos[:, None, :] == t_sub)
    jabs = jmat + b * T
    acc = jnp.sum(jnp.where(onehot, jabs[:, None, :], 0), axis=0)
    out_ref[...] = acc


def _run_select(e2, et2, xp, xcs):
    grid = (B, T4 // KB)
    qn = xcs[0].shape[1]
    in_specs = [
        pl.BlockSpec((KB, 1), lambda b, kb: (kb, 0)),
        pl.BlockSpec((1, KB), lambda b, kb: (0, kb)),
        pl.BlockSpec((1, T, D), lambda b, kb: (b, 0, 0)),
    ] + [
        pl.BlockSpec((1, qn, D), lambda b, kb: (b, 0, 0))
        for _ in range(4)
    ]
    return pl.pallas_call(
        _select_kernel,
        grid=grid,
        in_specs=in_specs,
        out_specs=pl.BlockSpec((W, KB), lambda b, kb: (0, b * (T4 // KB) + kb)),
        out_shape=jax.ShapeDtypeStruct((W, B * T4), jnp.int32),
    )(e2, et2, xp, *xcs)


_SC_CHUNK = 128
_SC_NC = 2
_SC_NS = 16
_SC_NW = _SC_NC * _SC_NS


def _sc_gather_body(table_hbm, idx_hbm, out_hbm, idx_v, r0, r1, gs0, gs1, ss0, ss1):
    nrows = B * T4 * W
    per_w = nrows // _SC_NW
    nchunk = per_w // _SC_CHUNK
    wid = lax.axis_index("s") * _SC_NC + lax.axis_index("c")
    base = wid * per_w
    bufs, gsems, ssems = (r0, r1), (gs0, gs1), (ss0, ss1)
    scat = [None, None]
    pltpu.sync_copy(idx_hbm.at[pl.ds(base, per_w)], idx_v)
    for ci in range(nchunk):
        s = ci & 1
        if scat[s] is not None:
            scat[s].wait()
        idx_c = idx_v.at[pl.ds(ci * _SC_CHUNK, _SC_CHUNK)]
        pltpu.async_copy(table_hbm.at[idx_c], bufs[s], gsems[s]).wait()
        scat[s] = pltpu.async_copy(
            bufs[s], out_hbm.at[pl.ds(base + ci * _SC_CHUNK, _SC_CHUNK)], ssems[s])
    scat[0].wait()
    scat[1].wait()


def _sc_gather(xflat, idxflat):
    nrows = B * T4 * W
    per_w = nrows // _SC_NW
    mesh = plsc.VectorSubcoreMesh(core_axis_name="c", subcore_axis_name="s")
    k = functools.partial(
        pl.kernel,
        mesh=mesh,
        out_type=jax.ShapeDtypeStruct((nrows, D), jnp.float32),
        scratch_types=[
            pltpu.VMEM((per_w,), jnp.int32),
            pltpu.VMEM((_SC_CHUNK, D), jnp.float32),
            pltpu.VMEM((_SC_CHUNK, D), jnp.float32),
            pltpu.SemaphoreType.DMA,
            pltpu.SemaphoreType.DMA,
            pltpu.SemaphoreType.DMA,
            pltpu.SemaphoreType.DMA,
        ],
    )(_sc_gather_body)
    return k(xflat, idxflat)


def _gru_kernel(f_ref, wih_ref, whh_ref, bih_ref, bhh_ref, out_ref):
    wih = wih_ref[...]
    whh = whh_ref[...]
    bih = bih_ref[...]
    bhh = bhh_ref[...]
    h = jnp.zeros((NB, D), jnp.float32)
    for t in range(W):
        xt = f_ref[t].astype(jnp.bfloat16)
        gi = jnp.dot(xt, wih, preferred_element_type=jnp.float32) + bih
        gh = jnp.dot(h.astype(jnp.bfloat16), whh,
                     preferred_element_type=jnp.float32) + bhh
        r = jax.nn.sigmoid(gi[:, :D] + gh[:, :D])
        z = jax.nn.sigmoid(gi[:, D:2 * D] + gh[:, D:2 * D])
        n = jnp.tanh(gi[:, 2 * D:] + r * gh[:, 2 * D:])
        h = (1.0 - z) * n + z * h
    out_ref[...] = h


def _run_gru(feat, wihT, whhT, bih2, bhh2):
    ntot = B * T4
    grid = (ntot // NB,)
    return pl.pallas_call(
        _gru_kernel,
        grid=grid,
        in_specs=[
            pl.BlockSpec((W, NB, D), lambda nb: (0, nb, 0)),
            pl.BlockSpec((D, 3 * D), lambda nb: (0, 0)),
            pl.BlockSpec((D, 3 * D), lambda nb: (0, 0)),
            pl.BlockSpec((1, 3 * D), lambda nb: (0, 0)),
            pl.BlockSpec((1, 3 * D), lambda nb: (0, 0)),
        ],
        out_specs=pl.BlockSpec((NB, D), lambda nb: (nb, 0)),
        out_shape=jax.ShapeDtypeStruct((ntot, D), jnp.float32),
    )(feat, wihT, whhT, bih2, bhh2)


def kernel(x, W_ih, W_hh, b_ih, b_hh):
    xr = x.reshape(B, T4, 4, D)
    xcs = [xr[:, :, c, :] for c in range(4)]
    e2 = jnp.asarray(_E).reshape(T4, 1)
    et2 = jnp.asarray(_E).reshape(1, T4)

    idx = _run_select(e2, et2, x, xcs)

    idx_t_major = idx.reshape(W * B * T4)
    xflat = x.reshape(B * T, D)
    feat = _sc_gather(xflat, idx_t_major)
    feat = feat.reshape(W, B * T4, D)

    h = _run_gru(feat, W_ih.T.astype(jnp.bfloat16), W_hh.T.astype(jnp.bfloat16),
                 b_ih.reshape(1, 3 * D), b_hh.reshape(1, 3 * D))
    return h.reshape(B, T4, D)


# speedup vs baseline: 1.1198x; 1.0322x over previous
import functools

import jax
import jax.numpy as jnp
import numpy as np
from jax import lax
from jax.experimental import pallas as pl
from jax.experimental.pallas import tpu as pltpu
from jax.experimental.pallas import tpu_sc as plsc

B, T, D = 4, 4096, 128
T4 = T // 4
W = 12
NW_CAND = 26
KB = 512
NB = 1024
NEG = -1e9

_TID = np.array(sorted(int(v) for v in np.linspace(0, T - 1, T4)), dtype=np.int32)
_E = (_TID - 4 * np.arange(T4, dtype=np.int32)).astype(np.int32)


NCOL = 4 * KB + 32


def _select_kernel(e_ref, et_ref, x_ref, x0_ref, x1_ref, x2_ref, x3_ref, out_ref):
    b = pl.program_id(0)
    kb = pl.program_id(1)
    xc = (x0_ref, x1_ref, x2_ref, x3_ref)

    e = e_ref[...]

    base_q = kb * KB
    sel_x = jnp.zeros((KB, D), jnp.float32)
    for c in range(4):
        seg = xc[c][0, pl.ds(base_q, KB), :]
        sel_x = jnp.where(e == c, seg, sel_x)

    KS = 32
    NCS = 4 * KS + 32
    lane_c = lax.broadcasted_iota(jnp.int32, (KS, NCS), 1)
    kl = lax.broadcasted_iota(jnp.int32, (KS, 1), 0)
    s_sub_rows = []
    for ks in range(0, KB, KS):
        sel_sub = sel_x[ks:ks + KS, :]
        n0 = 4 * (base_q + ks) - 16
        a0 = lax.clamp(0, n0, T - NCS)
        cols = x_ref[0, pl.ds(a0, NCS), :]
        S_sub = lax.dot_general(sel_sub, cols, (((1,), (1,)), ((), ())))
        off = n0 - a0
        sw = []
        for w in range(NW_CAND):
            m = lane_c == (4 * kl + (w + 5) + off)
            sw.append(jnp.sum(jnp.where(m, S_sub, 0.0), axis=1, keepdims=True))
        s_sub_rows.append(jnp.concatenate(sw, axis=1))
    Sraw = jnp.concatenate(s_sub_rows, axis=0)

    St = jnp.transpose(Sraw)
    eT = et_ref[...]
    k_lane = lax.broadcasted_iota(jnp.int32, (NW_CAND, KB), 1) + kb * KB
    w_sub = lax.broadcasted_iota(jnp.int32, (NW_CAND, KB), 0)
    jmat = 4 * k_lane + w_sub - 11
    valid = (w_sub >= eT) & (w_sub <= eT + 22) & (jmat >= 0) & (jmat <= T - 1)
    S = jnp.where(valid, St, NEG)

    Sp = S[:, None, :]
    Sq = S[None, :, :]
    wp = lax.broadcasted_iota(jnp.int32, (NW_CAND, NW_CAND, 1), 0)
    wq = lax.broadcasted_iota(jnp.int32, (NW_CAND, NW_CAND, 1), 1)
    beats = (Sp > Sq) | ((Sp == Sq) & (wp < wq))
    rank = jnp.sum(beats.astype(jnp.int32), axis=0)
    keep = rank < W
    pos = jnp.sum((keep[:, None, :] & (wp < wq)).astype(jnp.int32), axis=0)

    t_sub = lax.broadcasted_iota(jnp.int32, (NW_CAND, W, 1), 1)
    onehot = keep[:, None, :] & (pos[:, None, :] == t_sub)
    jabs = jmat + b * T
    acc = jnp.sum(jnp.where(onehot, jabs[:, None, :], 0), axis=0)
    out_ref[...] = acc


def _run_select(e2, et2, xp, xcs):
    grid = (B, T4 // KB)
    qn = xcs[0].shape[1]
    in_specs = [
        pl.BlockSpec((KB, 1), lambda b, kb: (kb, 0)),
        pl.BlockSpec((1, KB), lambda b, kb: (0, kb)),
        pl.BlockSpec((1, T, D), lambda b, kb: (b, 0, 0)),
    ] + [
        pl.BlockSpec((1, qn, D), lambda b, kb: (b, 0, 0))
        for _ in range(4)
    ]
    return pl.pallas_call(
        _select_kernel,
        grid=grid,
        in_specs=in_specs,
        out_specs=pl.BlockSpec((W, KB), lambda b, kb: (0, b * (T4 // KB) + kb)),
        out_shape=jax.ShapeDtypeStruct((W, B * T4), jnp.int32),
    )(e2, et2, xp, *xcs)


_SC_CHUNK = 128
_SC_NC = 2
_SC_NS = 16
_SC_NW = _SC_NC * _SC_NS


def _sc_gather_body(table_hbm, idx_hbm, out_hbm, idx_v, r0, r1, gs0, gs1, ss0, ss1,
                    *, nrows):
    per_w = nrows // _SC_NW
    nchunk = per_w // _SC_CHUNK
    wid = lax.axis_index("s") * _SC_NC + lax.axis_index("c")
    base = wid * per_w
    bufs, gsems, ssems = (r0, r1), (gs0, gs1), (ss0, ss1)
    scat = [None, None]
    pltpu.sync_copy(idx_hbm.at[pl.ds(base, per_w)], idx_v)
    for ci in range(nchunk):
        s = ci & 1
        if scat[s] is not None:
            scat[s].wait()
        idx_c = idx_v.at[pl.ds(ci * _SC_CHUNK, _SC_CHUNK)]
        pltpu.async_copy(table_hbm.at[idx_c], bufs[s], gsems[s]).wait()
        scat[s] = pltpu.async_copy(
            bufs[s], out_hbm.at[pl.ds(base + ci * _SC_CHUNK, _SC_CHUNK)], ssems[s])
    scat[0].wait()
    scat[1].wait()


def _sc_gather(xflat, idxflat):
    nrows = idxflat.shape[0]
    per_w = nrows // _SC_NW
    mesh = plsc.VectorSubcoreMesh(core_axis_name="c", subcore_axis_name="s")
    k = functools.partial(
        pl.kernel,
        mesh=mesh,
        out_type=jax.ShapeDtypeStruct((nrows, D), jnp.float32),
        scratch_types=[
            pltpu.VMEM((per_w,), jnp.int32),
            pltpu.VMEM((_SC_CHUNK, D), jnp.float32),
            pltpu.VMEM((_SC_CHUNK, D), jnp.float32),
            pltpu.SemaphoreType.DMA,
            pltpu.SemaphoreType.DMA,
            pltpu.SemaphoreType.DMA,
            pltpu.SemaphoreType.DMA,
        ],
    )(functools.partial(_sc_gather_body, nrows=nrows))
    return k(xflat, idxflat)


def _gru_kernel(f_ref, h0_ref, wih_ref, whh_ref, bih_ref, bhh_ref, out_ref,
                *, steps):
    wih = wih_ref[...]
    whh = whh_ref[...]
    bih = bih_ref[...]
    bhh = bhh_ref[...]
    h = h0_ref[...]
    for t in range(steps):
        xt = f_ref[t].astype(jnp.bfloat16)
        gi = jnp.dot(xt, wih, preferred_element_type=jnp.float32) + bih
        gh = jnp.dot(h.astype(jnp.bfloat16), whh,
                     preferred_element_type=jnp.float32) + bhh
        r = jax.nn.sigmoid(gi[:, :D] + gh[:, :D])
        z = jax.nn.sigmoid(gi[:, D:2 * D] + gh[:, D:2 * D])
        n = jnp.tanh(gi[:, 2 * D:] + r * gh[:, 2 * D:])
        h = (1.0 - z) * n + z * h
    out_ref[...] = h


def _run_gru(feat, h0, wihT, whhT, bih2, bhh2):
    ntot = B * T4
    steps = feat.shape[0]
    grid = (ntot // NB,)
    return pl.pallas_call(
        functools.partial(_gru_kernel, steps=steps),
        grid=grid,
        in_specs=[
            pl.BlockSpec((steps, NB, D), lambda nb: (0, nb, 0)),
            pl.BlockSpec((NB, D), lambda nb: (nb, 0)),
            pl.BlockSpec((D, 3 * D), lambda nb: (0, 0)),
            pl.BlockSpec((D, 3 * D), lambda nb: (0, 0)),
            pl.BlockSpec((1, 3 * D), lambda nb: (0, 0)),
            pl.BlockSpec((1, 3 * D), lambda nb: (0, 0)),
        ],
        out_specs=pl.BlockSpec((NB, D), lambda nb: (nb, 0)),
        out_shape=jax.ShapeDtypeStruct((ntot, D), jnp.float32),
    )(feat, h0, wihT, whhT, bih2, bhh2)


def kernel(x, W_ih, W_hh, b_ih, b_hh):
    xr = x.reshape(B, T4, 4, D)
    xcs = [xr[:, :, c, :] for c in range(4)]
    e2 = jnp.asarray(_E).reshape(T4, 1)
    et2 = jnp.asarray(_E).reshape(1, T4)

    idx = _run_select(e2, et2, x, xcs)

    idx_t_major = idx.reshape(W * B * T4)
    xflat = x.reshape(B * T, D)
    half = (W // 2) * B * T4
    wihT = W_ih.T.astype(jnp.bfloat16)
    whhT = W_hh.T.astype(jnp.bfloat16)
    bih2 = b_ih.reshape(1, 3 * D)
    bhh2 = b_hh.reshape(1, 3 * D)
    featA = _sc_gather(xflat, idx_t_major[:half]).reshape(W // 2, B * T4, D)
    featB = _sc_gather(xflat, idx_t_major[half:]).reshape(W // 2, B * T4, D)
    h0 = jnp.zeros((B * T4, D), jnp.float32)
    hA = _run_gru(featA, h0, wihT, whhT, bih2, bhh2)
    h = _run_gru(featB, hA, wihT, whhT, bih2, bhh2)
    return h.reshape(B, T4, D)


# speedup vs baseline: 1.1216x; 1.0016x over previous
import functools

import jax
import jax.numpy as jnp
import numpy as np
from jax import lax
from jax.experimental import pallas as pl
from jax.experimental.pallas import tpu as pltpu
from jax.experimental.pallas import tpu_sc as plsc

B, T, D = 4, 4096, 128
T4 = T // 4
W = 12
NW_CAND = 26
KB = 512
NB = 1024
NEG = -1e9

_TID = np.array(sorted(int(v) for v in np.linspace(0, T - 1, T4)), dtype=np.int32)
_E = (_TID - 4 * np.arange(T4, dtype=np.int32)).astype(np.int32)


def _select_kernel(e_ref, et_ref, x_ref, x0_ref, x1_ref, x2_ref, x3_ref, out_ref):
    b = pl.program_id(0)
    kb = pl.program_id(1)
    xc = (x0_ref, x1_ref, x2_ref, x3_ref)

    e = e_ref[...]

    base_q = kb * KB
    sel_x = jnp.zeros((KB, D), jnp.float32)
    for c in range(4):
        seg = xc[c][0, pl.ds(base_q, KB), :]
        sel_x = jnp.where(e == c, seg, sel_x)

    KS = 32
    NCS = 4 * KS + 32
    lane_c = lax.broadcasted_iota(jnp.int32, (KS, NCS), 1)
    kl = lax.broadcasted_iota(jnp.int32, (KS, 1), 0)
    s_sub_rows = []
    for ks in range(0, KB, KS):
        sel_sub = sel_x[ks:ks + KS, :]
        n0 = 4 * (base_q + ks) - 16
        a0 = lax.clamp(0, n0, T - NCS)
        cols = x_ref[0, pl.ds(a0, NCS), :]
        S_sub = lax.dot_general(sel_sub, cols, (((1,), (1,)), ((), ())))
        off = n0 - a0
        sw = []
        for w in range(NW_CAND):
            m = lane_c == (4 * kl + (w + 5) + off)
            sw.append(jnp.sum(jnp.where(m, S_sub, 0.0), axis=1, keepdims=True))
        s_sub_rows.append(jnp.concatenate(sw, axis=1))
    Sraw = jnp.concatenate(s_sub_rows, axis=0)

    St = jnp.transpose(Sraw)
    eT = et_ref[...]
    k_lane = lax.broadcasted_iota(jnp.int32, (NW_CAND, KB), 1) + kb * KB
    w_sub = lax.broadcasted_iota(jnp.int32, (NW_CAND, KB), 0)
    jmat = 4 * k_lane + w_sub - 11
    valid = (w_sub >= eT) & (w_sub <= eT + 22) & (jmat >= 0) & (jmat <= T - 1)
    S = jnp.where(valid, St, NEG)

    Sp = S[:, None, :]
    Sq = S[None, :, :]
    wp = lax.broadcasted_iota(jnp.int32, (NW_CAND, NW_CAND, 1), 0)
    wq = lax.broadcasted_iota(jnp.int32, (NW_CAND, NW_CAND, 1), 1)
    beats = (Sp > Sq) | ((Sp == Sq) & (wp < wq))
    rank = jnp.sum(beats.astype(jnp.int32), axis=0)
    keep = rank < W
    pos = jnp.sum((keep[:, None, :] & (wp < wq)).astype(jnp.int32), axis=0)

    t_sub = lax.broadcasted_iota(jnp.int32, (NW_CAND, W, 1), 1)
    onehot = keep[:, None, :] & (pos[:, None, :] == t_sub)
    jabs = jmat + b * T
    acc = jnp.sum(jnp.where(onehot, jabs[:, None, :], 0), axis=0)
    out_ref[...] = acc


def _run_select(e2, et2, xp, xcs):
    grid = (B, T4 // KB)
    qn = xcs[0].shape[1]
    in_specs = [
        pl.BlockSpec((KB, 1), lambda b, kb: (kb, 0)),
        pl.BlockSpec((1, KB), lambda b, kb: (0, kb)),
        pl.BlockSpec((1, T, D), lambda b, kb: (b, 0, 0)),
    ] + [
        pl.BlockSpec((1, qn, D), lambda b, kb: (b, 0, 0))
        for _ in range(4)
    ]
    return pl.pallas_call(
        _select_kernel,
        grid=grid,
        in_specs=in_specs,
        out_specs=pl.BlockSpec((W, KB), lambda b, kb: (0, b * (T4 // KB) + kb)),
        out_shape=jax.ShapeDtypeStruct((W, B * T4), jnp.int32),
    )(e2, et2, xp, *xcs)


_SC_CHUNK = 128
_SC_NC = 2
_SC_NS = 16
_SC_NW = _SC_NC * _SC_NS


def _sc_gather_body(table_hbm, idx_hbm, out_hbm, idx_v, r0, r1, gs0, gs1, ss0, ss1,
                    *, nrows):
    per_w = nrows // _SC_NW
    nchunk = per_w // _SC_CHUNK
    wid = lax.axis_index("s") * _SC_NC + lax.axis_index("c")
    base = wid * per_w
    bufs, gsems, ssems = (r0, r1), (gs0, gs1), (ss0, ss1)
    scat = [None, None]
    pltpu.sync_copy(idx_hbm.at[pl.ds(base, per_w)], idx_v)
    for ci in range(nchunk):
        s = ci & 1
        if scat[s] is not None:
            scat[s].wait()
        idx_c = idx_v.at[pl.ds(ci * _SC_CHUNK, _SC_CHUNK)]
        pltpu.async_copy(table_hbm.at[idx_c], bufs[s], gsems[s]).wait()
        scat[s] = pltpu.async_copy(
            bufs[s], out_hbm.at[pl.ds(base + ci * _SC_CHUNK, _SC_CHUNK)], ssems[s])
    scat[0].wait()
    scat[1].wait()


def _sc_gather(xflat, idxflat):
    nrows = idxflat.shape[0]
    per_w = nrows // _SC_NW
    mesh = plsc.VectorSubcoreMesh(core_axis_name="c", subcore_axis_name="s")
    k = functools.partial(
        pl.kernel,
        mesh=mesh,
        out_type=jax.ShapeDtypeStruct((nrows, D), jnp.float32),
        scratch_types=[
            pltpu.VMEM((per_w,), jnp.int32),
            pltpu.VMEM((_SC_CHUNK, D), jnp.float32),
            pltpu.VMEM((_SC_CHUNK, D), jnp.float32),
            pltpu.SemaphoreType.DMA,
            pltpu.SemaphoreType.DMA,
            pltpu.SemaphoreType.DMA,
            pltpu.SemaphoreType.DMA,
        ],
    )(functools.partial(_sc_gather_body, nrows=nrows))
    return k(xflat, idxflat)


def _gru_kernel(f_ref, h0_ref, wih_ref, whh_ref, bih_ref, bhh_ref, out_ref,
                *, steps):
    wih = wih_ref[...]
    whh = whh_ref[...]
    bih = bih_ref[...]
    bhh = bhh_ref[...]
    h = h0_ref[...]
    for t in range(steps):
        xt = f_ref[t].astype(jnp.bfloat16)
        gi = jnp.dot(xt, wih, preferred_element_type=jnp.float32) + bih
        gh = jnp.dot(h.astype(jnp.bfloat16), whh,
                     preferred_element_type=jnp.float32) + bhh
        r = jax.nn.sigmoid(gi[:, :D] + gh[:, :D])
        z = jax.nn.sigmoid(gi[:, D:2 * D] + gh[:, D:2 * D])
        n = jnp.tanh(gi[:, 2 * D:] + r * gh[:, 2 * D:])
        h = (1.0 - z) * n + z * h
    out_ref[...] = h


def _run_gru(feat, h0, wihT, whhT, bih2, bhh2):
    ntot = B * T4
    steps = feat.shape[0]
    grid = (ntot // NB,)
    return pl.pallas_call(
        functools.partial(_gru_kernel, steps=steps),
        grid=grid,
        in_specs=[
            pl.BlockSpec((steps, NB, D), lambda nb: (0, nb, 0)),
            pl.BlockSpec((NB, D), lambda nb: (nb, 0)),
            pl.BlockSpec((D, 3 * D), lambda nb: (0, 0)),
            pl.BlockSpec((D, 3 * D), lambda nb: (0, 0)),
            pl.BlockSpec((1, 3 * D), lambda nb: (0, 0)),
            pl.BlockSpec((1, 3 * D), lambda nb: (0, 0)),
        ],
        out_specs=pl.BlockSpec((NB, D), lambda nb: (nb, 0)),
        out_shape=jax.ShapeDtypeStruct((ntot, D), jnp.float32),
    )(feat, h0, wihT, whhT, bih2, bhh2)


def kernel(x, W_ih, W_hh, b_ih, b_hh):
    xr = x.reshape(B, T4, 4, D)
    xcs = [xr[:, :, c, :] for c in range(4)]
    e2 = jnp.asarray(_E).reshape(T4, 1)
    et2 = jnp.asarray(_E).reshape(1, T4)

    idx = _run_select(e2, et2, x, xcs)

    idx_t_major = idx.reshape(W * B * T4)
    xflat = x.reshape(B * T, D)
    half = (W // 2) * B * T4
    wihT = W_ih.T.astype(jnp.bfloat16)
    whhT = W_hh.T.astype(jnp.bfloat16)
    bih2 = b_ih.reshape(1, 3 * D)
    bhh2 = b_hh.reshape(1, 3 * D)
    featA = _sc_gather(xflat, idx_t_major[:half]).reshape(W // 2, B * T4, D)
    featB = _sc_gather(xflat, idx_t_major[half:]).reshape(W // 2, B * T4, D)
    h0 = jnp.zeros((B * T4, D), jnp.float32)
    hA = _run_gru(featA, h0, wihT, whhT, bih2, bhh2)
    h = _run_gru(featB, hA, wihT, whhT, bih2, bhh2)
    return h.reshape(B, T4, D)
